# baseline (speedup 1.0000x reference)
import functools

import jax
import jax.numpy as jnp
from jax import lax
from jax.experimental import pallas as pl
from jax.experimental.pallas import tpu as pltpu
from jax.experimental.pallas import tpu_sc as plsc

EMBED = 64
NC = 2
NS = 16
NW = NC * NS
CHUNK = 128
NBUF = 8
LOOK = 4

def _sel(n, c0, cols):
    ii = lax.broadcasted_iota(jnp.int32, (n, cols), 0)
    jj = lax.broadcasted_iota(jnp.int32, (n, cols), 1)
    return (ii == jj + c0).astype(jnp.float32)


def _transpose_out(g2, s0, nseq, seq, b, emb, prev=None):
    half = b // 2

    def body(*refs):
        x_ref, o_ref = refs[0], refs[-1]
        z = lax.dot_general(
            _sel(2 * emb, 0, 2 * emb), x_ref[...], (((1,), (1,)), ((), ())),
            preferred_element_type=jnp.float32,
        )
        o_ref[0, :, :half] = z[:emb, :]
        o_ref[0, :, half:] = z[emb:, :]

    in_specs = [pl.BlockSpec((half, 2 * emb), lambda j: (j, 0))]
    args = [g2]
    kwargs = {}
    if prev is not None:
        in_specs.append(pl.BlockSpec((1, 8, 128), lambda j: (0, 0, 0)))
        args.append(prev)
        kwargs["input_output_aliases"] = {1: 0}
    return pl.pallas_call(
        body,
        grid=(nseq,),
        in_specs=in_specs,
        out_specs=pl.BlockSpec((1, emb, b), lambda j: (s0 + j, 0, 0)),
        out_shape=jax.ShapeDtypeStruct((seq, emb, b), jnp.float32),
        **kwargs,
    )(*args)


def _make_gather(n_total, row_off=0):
    per_w = n_total // NW
    nch = per_w // CHUNK
    ngroups = nch // NBUF
    assert nch % NBUF == 0 and ngroups >= 3
    mesh = plsc.VectorSubcoreMesh(core_axis_name="c", subcore_axis_name="s")

    @functools.partial(
        pl.kernel,
        mesh=mesh,
        out_type=jax.ShapeDtypeStruct((n_total, EMBED), jnp.float32),
        compiler_params=pltpu.CompilerParams(use_tc_tiling_on_sc=False),
        scratch_types=[
            pltpu.VMEM((nch, CHUNK), jnp.int32),
            pltpu.VMEM((NBUF, CHUNK, EMBED), jnp.float32),
            pltpu.SemaphoreType.DMA((NBUF,)),
            pltpu.SemaphoreType.DMA((NBUF,)),
        ],
    )
    def gather_kernel(table_hbm, idx_hbm, out_hbm, idx_v, rows_v, gsem, wsem):
        wid = lax.axis_index("s") * NC + lax.axis_index("c")
        base = wid * per_w
        pltpu.sync_copy(idx_hbm.at[pl.ds(row_off + wid * nch, nch)], idx_v)

        def gather_start(j, b):
            pltpu.async_copy(table_hbm.at[idx_v.at[j]], rows_v.at[b], gsem.at[b])

        def gather_wait(j, b):
            pltpu.make_async_copy(
                table_hbm.at[idx_v.at[j]], rows_v.at[b], gsem.at[b]
            ).wait()

        def wb_start(j, b):
            pltpu.async_copy(
                rows_v.at[b],
                out_hbm.at[pl.ds(base + j * CHUNK, CHUNK)],
                wsem.at[b],
            )

        def wb_wait(j, b):
            pltpu.make_async_copy(
                rows_v.at[b],
                out_hbm.at[pl.ds(base + j * CHUNK, CHUNK)],
                wsem.at[b],
            ).wait()

        for j in range(LOOK):
            gather_start(j, j % NBUF)

        def step(j, b, bn, first_group):
            jn = j + LOOK
            if not (first_group and jn < NBUF):
                wb_wait(jn - NBUF, bn)
            gather_start(jn, bn)
            gather_wait(j, b)
            wb_start(j, b)

        for b in range(NBUF):
            step(b, b, (b + LOOK) % NBUF, True)

        def group(g, carry):
            j0 = g * NBUF
            for b in range(NBUF):
                step(j0 + b, b, (b + LOOK) % NBUF, False)
            return carry

        lax.fori_loop(1, ngroups - 1, group, 0)

        j0 = (ngroups - 1) * NBUF
        for b in range(NBUF):
            j = j0 + b
            jn = j + LOOK
            bn = (b + LOOK) % NBUF
            if jn < nch:
                wb_wait(jn - NBUF, bn)
                gather_start(jn, bn)
            gather_wait(j, b)
            wb_start(j, b)

        for j in range(nch - NBUF, nch):
            wb_wait(j, j % NBUF)

    return gather_kernel


def kernel(input_x, table):
    batch, seq = input_x.shape
    n = batch * seq
    half = batch // 2
    idx_t = input_x.T.astype(jnp.int32)
    idx_p = (
        idx_t.reshape(seq, 2, half).transpose(0, 2, 1).reshape(seq, batch)
    )
    idx2 = idx_p.reshape(n // CHUNK, CHUNK)

    nsplit = 5
    s_per = seq // nsplit
    n_per = s_per * batch
    rows_per = n_per // CHUNK
    gather = [
        _make_gather(n_per, row_off=i * rows_per)(table, idx2)
        for i in range(nsplit)
    ]
    out3 = None
    for i in range(nsplit):
        out3 = _transpose_out(
            gather[i].reshape(n_per // 2, 2 * EMBED),
            i * s_per, s_per, seq, batch, EMBED, prev=out3,
        )
    return jnp.transpose(out3, (2, 0, 1))


# speedup vs baseline: 1.4878x; 1.4878x over previous
import functools

import jax
import jax.numpy as jnp
from jax import lax
from jax.experimental import pallas as pl
from jax.experimental.pallas import tpu as pltpu
from jax.experimental.pallas import tpu_sc as plsc

EMBED = 64
NC = 2
NS = 16
NW = NC * NS
CHUNK = 128
NBUF = 5
LOOK = 2

TBLK = 4096
NSUP = 62
V4 = NSUP * TBLK


def _sel(n, c0, cols):
    ii = lax.broadcasted_iota(jnp.int32, (n, cols), 0)
    jj = lax.broadcasted_iota(jnp.int32, (n, cols), 1)
    return (ii == jj + c0).astype(jnp.float32)


def _transpose_table(table_t):
    emb, v = table_t.shape

    def body(x_ref, o_ref):
        x = x_ref[...]
        l = lax.concatenate(
            [x[:, k * TBLK:(k + 1) * TBLK] for k in range(4)], 0
        )
        o_ref[:TBLK] = lax.dot_general(
            l, _sel(4 * emb, 0, 2 * emb), (((0,), (0,)), ((), ())),
            preferred_element_type=jnp.float32,
        )
        o_ref[TBLK:] = lax.dot_general(
            l, _sel(4 * emb, 2 * emb, 2 * emb), (((0,), (0,)), ((), ())),
            preferred_element_type=jnp.float32,
        )

    return pl.pallas_call(
        body,
        grid=(NSUP,),
        in_specs=[pl.BlockSpec((emb, 4 * TBLK), lambda j: (0, j))],
        out_specs=pl.BlockSpec((2 * TBLK, 2 * emb), lambda j: (j, 0)),
        out_shape=jax.ShapeDtypeStruct((2 * V4, 2 * emb), jnp.float32),
    )(table_t)


def _transpose_out(g2, s0, nseq, seq, b, emb, prev=None):
    half = b // 2

    def body(*refs):
        x_ref, o_ref = refs[0], refs[-1]
        z = lax.dot_general(
            _sel(2 * emb, 0, 2 * emb), x_ref[...], (((1,), (1,)), ((), ())),
            preferred_element_type=jnp.float32,
        )
        o_ref[0, :, :half] = z[:emb, :]
        o_ref[0, :, half:] = z[emb:, :]

    in_specs = [pl.BlockSpec((half, 2 * emb), lambda j: (j, 0))]
    args = [g2]
    kwargs = {}
    if prev is not None:
        in_specs.append(pl.BlockSpec((1, 8, 128), lambda j: (0, 0, 0)))
        args.append(prev)
        kwargs["input_output_aliases"] = {1: 0}
    return pl.pallas_call(
        body,
        grid=(nseq,),
        in_specs=in_specs,
        out_specs=pl.BlockSpec((1, emb, b), lambda j: (s0 + j, 0, 0)),
        out_shape=jax.ShapeDtypeStruct((seq, emb, b), jnp.float32),
        **kwargs,
    )(*args)


def _make_gather(n_total, row_off=0):
    per_w = n_total // NW
    nch = per_w // CHUNK
    ngroups = nch // NBUF
    assert nch % NBUF == 0 and ngroups >= 3
    mesh = plsc.VectorSubcoreMesh(core_axis_name="c", subcore_axis_name="s")

    @functools.partial(
        pl.kernel,
        mesh=mesh,
        out_type=jax.ShapeDtypeStruct((n_total, EMBED), jnp.float32),
        compiler_params=pltpu.CompilerParams(use_tc_tiling_on_sc=False),
        scratch_types=[
            pltpu.VMEM((nch, CHUNK), jnp.int32),
            pltpu.VMEM((NBUF, CHUNK, EMBED), jnp.float32),
            pltpu.SemaphoreType.DMA((NBUF,)),
            pltpu.SemaphoreType.DMA((NBUF,)),
        ],
    )
    def gather_kernel(table_hbm, idx_hbm, out_hbm, idx_v, rows_v, gsem, wsem):
        wid = lax.axis_index("s") * NC + lax.axis_index("c")
        base = wid * per_w
        pltpu.sync_copy(idx_hbm.at[pl.ds(row_off + wid * nch, nch)], idx_v)

        def remap_row(j):
            for k in range(CHUNK // 16):
                r = idx_v[j, pl.ds(16 * k, 16)]
                hi = lax.shift_left(lax.shift_right_logical(r, 14), 14)
                w = lax.shift_left(
                    lax.bitwise_and(lax.shift_right_logical(r, 13), 1), 13
                )
                v2 = lax.shift_left(lax.bitwise_and(r, 4095), 1)
                t = lax.bitwise_and(lax.shift_right_logical(r, 12), 1)
                idx_v[j, pl.ds(16 * k, 16)] = lax.bitwise_or(
                    lax.bitwise_or(hi, w), lax.bitwise_or(v2, t)
                )

        def gather_start(j, b):
            pltpu.async_copy(table_hbm.at[idx_v.at[j]], rows_v.at[b], gsem.at[b])

        def gather_wait(j, b):
            pltpu.make_async_copy(
                table_hbm.at[idx_v.at[j]], rows_v.at[b], gsem.at[b]
            ).wait()

        def wb_start(j, b):
            pltpu.async_copy(
                rows_v.at[b],
                out_hbm.at[pl.ds(base + j * CHUNK, CHUNK)],
                wsem.at[b],
            )

        def wb_wait(j, b):
            pltpu.make_async_copy(
                rows_v.at[b],
                out_hbm.at[pl.ds(base + j * CHUNK, CHUNK)],
                wsem.at[b],
            ).wait()

        for j in range(LOOK):
            remap_row(j)
            gather_start(j, j % NBUF)

        def step(j, b, bn, first_group):
            jn = j + LOOK
            if not (first_group and jn < NBUF):
                wb_wait(jn - NBUF, bn)
            remap_row(jn)
            gather_start(jn, bn)
            gather_wait(j, b)
            wb_start(j, b)

        for b in range(NBUF):
            step(b, b, (b + LOOK) % NBUF, True)

        def group(g, carry):
            j0 = g * NBUF
            for b in range(NBUF):
                step(j0 + b, b, (b + LOOK) % NBUF, False)
            return carry

        lax.fori_loop(1, ngroups - 1, group, 0)

        j0 = (ngroups - 1) * NBUF
        for b in range(NBUF):
            j = j0 + b
            jn = j + LOOK
            bn = (b + LOOK) % NBUF
            if jn < nch:
                wb_wait(jn - NBUF, bn)
                remap_row(jn)
                gather_start(jn, bn)
            gather_wait(j, b)
            wb_start(j, b)

        for j in range(nch - NBUF, nch):
            wb_wait(j, j % NBUF)

    return gather_kernel


def kernel(input_x, table):
    batch, seq = input_x.shape
    n = batch * seq
    half = batch // 2
    idx_t = input_x.T.astype(jnp.int32)
    idx_p = (
        idx_t.reshape(seq, 2, half).transpose(0, 2, 1).reshape(seq, batch)
    )
    idx2 = idx_p.reshape(n // CHUNK, CHUNK)
    table_packed = _transpose_table(table.T).reshape(4 * V4, EMBED)

    nsplit = 10
    s_per = seq // nsplit
    n_per = s_per * batch
    rows_per = n_per // CHUNK
    gather = [
        _make_gather(n_per, row_off=i * rows_per)(table_packed, idx2)
        for i in range(nsplit)
    ]
    out3 = None
    for i in range(nsplit):
        out3 = _transpose_out(
            gather[i].reshape(n_per // 2, 2 * EMBED),
            i * s_per, s_per, seq, batch, EMBED, prev=out3,
        )
    return jnp.transpose(out3, (2, 0, 1))
